# baseline (speedup 1.0000x reference)
import functools

import jax
import jax.numpy as jnp
from jax import lax
from jax.experimental import pallas as pl
from jax.experimental.pallas import tpu as pltpu


def _fused_block_kernel(x_ref, w1_ref, w2_ref, bn_ref, fc1_ref, fc2_ref,
                        o_ref, xs_ref, ys_ref, *,
                        chunk, H, W, C, inv_hw):
    f32 = jnp.float32
    bf16 = jnp.bfloat16
    HW = H * W
    S = HW + W
    M = chunk * S
    G = 32
    GB = 32

    bn1b, bn2b = bn_ref[0:1, :], bn_ref[1:2, :]

    r_id = lax.broadcasted_iota(jnp.int32, (M, 1), 0)
    w_id = jnp.remainder(r_id, W)
    m_notL = (w_id != (W - 1)).astype(bf16)
    m_notR = (w_id != 0).astype(bf16)

    xs_ref[0:G, :] = jnp.zeros((G, 3 * C), bf16)
    xs_ref[G + M:G + M + GB, :] = jnp.zeros((GB, 3 * C), bf16)
    for n in range(chunk):
        base = G + n * S
        xs_ref[base + HW:base + S, C:2 * C] = jnp.zeros((W, C), bf16)
        xs_ref[base:base + HW, C:2 * C] = x_ref[n].astype(bf16)
    xs_ref[G:G + M, 0:C] = xs_ref[G + 1:G + 1 + M, C:2 * C] * m_notL
    xs_ref[G:G + M, 2 * C:3 * C] = xs_ref[G - 1:G - 1 + M, C:2 * C] * m_notR

    acc = jnp.dot(xs_ref[G - W:G - W + M, :], w1_ref[0],
                  preferred_element_type=f32)
    acc += jnp.dot(xs_ref[G:G + M, :], w1_ref[1], preferred_element_type=f32)
    acc += jnp.dot(xs_ref[G + W:G + W + M, :], w1_ref[2],
                   preferred_element_type=f32)
    y1 = jnp.maximum(acc + bn1b, 0.0)

    ys_ref[0:G, :] = jnp.zeros((G, 3 * C), bf16)
    ys_ref[G + M:G + M + GB, :] = jnp.zeros((GB, 3 * C), bf16)
    ys_ref[G:G + M, C:2 * C] = y1.astype(bf16)
    for n in range(chunk):
        base = G + n * S
        ys_ref[base + HW:base + S, C:2 * C] = jnp.zeros((W, C), bf16)
    ys_ref[G:G + M, 0:C] = ys_ref[G + 1:G + 1 + M, C:2 * C] * m_notL
    ys_ref[G:G + M, 2 * C:3 * C] = ys_ref[G - 1:G - 1 + M, C:2 * C] * m_notR

    acc2 = jnp.dot(ys_ref[G - W:G - W + M, :], w2_ref[0],
                   preferred_element_type=f32)
    acc2 += jnp.dot(ys_ref[G:G + M, :], w2_ref[1], preferred_element_type=f32)
    acc2 += jnp.dot(ys_ref[G + W:G + W + M, :], w2_ref[2],
                    preferred_element_type=f32)
    r2 = acc2 + bn2b

    pooled = jnp.concatenate(
        [jnp.sum(r2[n * S:n * S + HW, :], axis=0, keepdims=True)
         for n in range(chunk)], axis=0)
    hid = jnp.maximum(jnp.dot(pooled, fc1_ref[...],
                              preferred_element_type=f32), 0.0)
    att = jax.nn.sigmoid(jnp.dot(hid, fc2_ref[...], preferred_element_type=f32))

    for n in range(chunk):
        seg = r2[n * S:n * S + HW, :] * att[n:n + 1, :]
        o_ref[n] = jnp.maximum(seg + x_ref[n], 0.0).astype(o_ref.dtype)


@jax.jit
def _basic_block_att(x_nchw, w1, bn1_scale, bn1_bias, w2, bn2_scale, bn2_bias,
                     fc1, fc2):
    N, C, H, W = x_nchw.shape
    HW = H * W
    S = HW + W
    bf16 = jnp.bfloat16

    chunk = next(c for c in (8, 4, 2, 1) if N % c == 0 and N // c >= 2)
    n_steps = N // chunk
    M = chunk * S

    x = jnp.transpose(x_nchw, (0, 2, 3, 1)).reshape(N, HW, C)

    def pack(w, scale):
        w = jnp.asarray(w, jnp.float32) * scale[None, None, None, :]
        return jnp.stack([
            jnp.concatenate([w[kh, 2], w[kh, 1], w[kh, 0]], axis=0)
            for kh in range(3)]).astype(bf16)

    w1p = pack(w1, jnp.asarray(bn1_scale, jnp.float32))
    w2p = pack(w2, jnp.asarray(bn2_scale, jnp.float32))
    bn = jnp.stack([bn1_bias, bn2_bias]).astype(jnp.float32)
    fc1 = jnp.asarray(fc1, jnp.float32) * (1.0 / float(HW))
    fc2 = jnp.asarray(fc2, jnp.float32)
    Cr = fc1.shape[1]

    kfn = functools.partial(_fused_block_kernel, chunk=chunk, H=H, W=W, C=C,
                            inv_hw=1.0 / float(HW))

    out = pl.pallas_call(
        kfn,
        out_shape=jax.ShapeDtypeStruct((N, HW, C), bf16),
        grid=(n_steps,),
        in_specs=[
            pl.BlockSpec((chunk, HW, C), lambda b: (b, 0, 0)),
            pl.BlockSpec((3, 3 * C, C), lambda b: (0, 0, 0)),
            pl.BlockSpec((3, 3 * C, C), lambda b: (0, 0, 0)),
            pl.BlockSpec((2, C), lambda b: (0, 0)),
            pl.BlockSpec((C, Cr), lambda b: (0, 0)),
            pl.BlockSpec((Cr, C), lambda b: (0, 0)),
        ],
        out_specs=pl.BlockSpec((chunk, HW, C), lambda b: (b, 0, 0)),
        scratch_shapes=[pltpu.VMEM((M + 64, 3 * C), bf16),
                        pltpu.VMEM((M + 64, 3 * C), bf16)],
        compiler_params=pltpu.CompilerParams(
            dimension_semantics=("parallel",),
            vmem_limit_bytes=48 * 1024 * 1024),
    )(x, w1p, w2p, bn, fc1, fc2)

    out = out.reshape(N, H, W, C)
    return jnp.transpose(out, (0, 3, 1, 2)).astype(jnp.float32)


def kernel(x, w1, bn1_scale, bn1_bias, w2, bn2_scale, bn2_bias,
           ws, bns_scale, bns_bias, fc1, fc2):
    del ws, bns_scale, bns_bias
    return _basic_block_att(x, w1, bn1_scale, bn1_bias,
                            w2, bn2_scale, bn2_bias, fc1, fc2)


# speedup vs baseline: 1.1185x; 1.1185x over previous
import functools

import jax
import jax.numpy as jnp
from jax import lax
from jax.experimental import pallas as pl
from jax.experimental.pallas import tpu as pltpu


def _fused_block_kernel(x_ref, w1_ref, w2_ref, bn_ref, fc1_ref, fc2_ref,
                        o_ref, xs_ref, ys_ref, *,
                        chunk, H, W, C, inv_hw):
    f32 = jnp.float32
    bf16 = jnp.bfloat16
    HW = H * W
    S = HW + W
    M = chunk * S
    G = 32
    GB = 32

    bn1s, bn1b = bn_ref[0:1, :], bn_ref[1:2, :]
    bn2s, bn2b = bn_ref[2:3, :], bn_ref[3:4, :]

    r_id = lax.broadcasted_iota(jnp.int32, (M, 1), 0)
    w_id = jnp.remainder(r_id, W)
    m_notL = (w_id != (W - 1)).astype(bf16)
    m_notR = (w_id != 0).astype(bf16)

    xs_ref[0:G, :] = jnp.zeros((G, 3 * C), bf16)
    xs_ref[G + M:G + M + GB, :] = jnp.zeros((GB, 3 * C), bf16)
    for n in range(chunk):
        base = G + n * S
        xs_ref[base + HW:base + S, C:2 * C] = jnp.zeros((W, C), bf16)
        xs_ref[base:base + HW, C:2 * C] = x_ref[n].astype(bf16)
    xs_ref[G:G + M, 0:C] = xs_ref[G + 1:G + 1 + M, C:2 * C] * m_notL
    xs_ref[G:G + M, 2 * C:3 * C] = xs_ref[G - 1:G - 1 + M, C:2 * C] * m_notR

    acc = jnp.dot(xs_ref[G - W:G - W + M, :], w1_ref[0],
                  preferred_element_type=f32)
    acc += jnp.dot(xs_ref[G:G + M, :], w1_ref[1], preferred_element_type=f32)
    acc += jnp.dot(xs_ref[G + W:G + W + M, :], w1_ref[2],
                   preferred_element_type=f32)
    y1 = jnp.maximum(acc * bn1s + bn1b, 0.0)

    ys_ref[0:G, :] = jnp.zeros((G, 3 * C), bf16)
    ys_ref[G + M:G + M + GB, :] = jnp.zeros((GB, 3 * C), bf16)
    ys_ref[G:G + M, C:2 * C] = y1.astype(bf16)
    for n in range(chunk):
        base = G + n * S
        ys_ref[base + HW:base + S, C:2 * C] = jnp.zeros((W, C), bf16)
    ys_ref[G:G + M, 0:C] = ys_ref[G + 1:G + 1 + M, C:2 * C] * m_notL
    ys_ref[G:G + M, 2 * C:3 * C] = ys_ref[G - 1:G - 1 + M, C:2 * C] * m_notR

    acc2 = jnp.dot(ys_ref[G - W:G - W + M, :], w2_ref[0],
                   preferred_element_type=f32)
    acc2 += jnp.dot(ys_ref[G:G + M, :], w2_ref[1], preferred_element_type=f32)
    acc2 += jnp.dot(ys_ref[G + W:G + W + M, :], w2_ref[2],
                    preferred_element_type=f32)
    r2 = acc2 * bn2s + bn2b

    pooled = jnp.concatenate(
        [jnp.sum(r2[n * S:n * S + HW, :], axis=0, keepdims=True)
         for n in range(chunk)], axis=0) * inv_hw
    hid = jnp.maximum(jnp.dot(pooled, fc1_ref[...],
                              preferred_element_type=f32), 0.0)
    att = jax.nn.sigmoid(jnp.dot(hid, fc2_ref[...], preferred_element_type=f32))

    for n in range(chunk):
        seg = r2[n * S:n * S + HW, :] * att[n:n + 1, :]
        o_ref[n] = jnp.maximum(seg + x_ref[n], 0.0)


@jax.jit
def _basic_block_att(x_nchw, w1, bn1_scale, bn1_bias, w2, bn2_scale, bn2_bias,
                     fc1, fc2):
    N, C, H, W = x_nchw.shape
    HW = H * W
    S = HW + W
    bf16 = jnp.bfloat16

    chunk = next(c for c in (8, 4, 2, 1) if N % c == 0 and N // c >= 2)
    n_steps = N // chunk
    M = chunk * S

    x = jnp.transpose(x_nchw, (0, 2, 3, 1)).reshape(N, HW, C)

    def pack(w):
        w = jnp.asarray(w)
        return jnp.stack([
            jnp.concatenate([w[kh, 2], w[kh, 1], w[kh, 0]], axis=0)
            for kh in range(3)]).astype(bf16)

    w1p = pack(w1)
    w2p = pack(w2)
    bn = jnp.stack([bn1_scale, bn1_bias, bn2_scale, bn2_bias]
                   ).astype(jnp.float32)
    fc1 = jnp.asarray(fc1, jnp.float32)
    fc2 = jnp.asarray(fc2, jnp.float32)
    Cr = fc1.shape[1]

    kfn = functools.partial(_fused_block_kernel, chunk=chunk, H=H, W=W, C=C,
                            inv_hw=1.0 / float(HW))

    out = pl.pallas_call(
        kfn,
        out_shape=jax.ShapeDtypeStruct((N, HW, C), jnp.float32),
        grid=(n_steps,),
        in_specs=[
            pl.BlockSpec((chunk, HW, C), lambda b: (b, 0, 0)),
            pl.BlockSpec((3, 3 * C, C), lambda b: (0, 0, 0)),
            pl.BlockSpec((3, 3 * C, C), lambda b: (0, 0, 0)),
            pl.BlockSpec((4, C), lambda b: (0, 0)),
            pl.BlockSpec((C, Cr), lambda b: (0, 0)),
            pl.BlockSpec((Cr, C), lambda b: (0, 0)),
        ],
        out_specs=pl.BlockSpec((chunk, HW, C), lambda b: (b, 0, 0)),
        scratch_shapes=[pltpu.VMEM((M + 64, 3 * C), bf16),
                        pltpu.VMEM((M + 64, 3 * C), bf16)],
        compiler_params=pltpu.CompilerParams(
            dimension_semantics=("parallel",),
            vmem_limit_bytes=48 * 1024 * 1024),
    )(x, w1p, w2p, bn, fc1, fc2)

    out = out.reshape(N, H, W, C)
    return jnp.transpose(out, (0, 3, 1, 2))


def kernel(x, w1, bn1_scale, bn1_bias, w2, bn2_scale, bn2_bias,
           ws, bns_scale, bns_bias, fc1, fc2):
    del ws, bns_scale, bns_bias
    return _basic_block_att(x, w1, bn1_scale, bn1_bias,
                            w2, bn2_scale, bn2_bias, fc1, fc2)


# speedup vs baseline: 1.1586x; 1.0358x over previous
import functools

import jax
import jax.numpy as jnp
from jax import lax
from jax.experimental import pallas as pl
from jax.experimental.pallas import tpu as pltpu


def _fused_block_kernel(x_ref, w1_ref, w2_ref, bn_ref, fc1_ref, fc2_ref,
                        o_ref, xs_ref, *,
                        chunk, H, W, C, inv_hw):
    ys_ref = xs_ref
    f32 = jnp.float32
    bf16 = jnp.bfloat16
    HW = H * W
    S = HW + W
    M = chunk * S
    G = 32
    GB = 32

    bn1s, bn1b = bn_ref[0:1, :], bn_ref[1:2, :]
    bn2s, bn2b = bn_ref[2:3, :], bn_ref[3:4, :]

    r_id = lax.broadcasted_iota(jnp.int32, (M, 1), 0)
    w_id = jnp.remainder(r_id, W)
    m_notL = (w_id != (W - 1)).astype(bf16)
    m_notR = (w_id != 0).astype(bf16)

    xs_ref[0:G, :] = jnp.zeros((G, 3 * C), bf16)
    xs_ref[G + M:G + M + GB, :] = jnp.zeros((GB, 3 * C), bf16)
    for n in range(chunk):
        base = G + n * S
        xs_ref[base + HW:base + S, C:2 * C] = jnp.zeros((W, C), bf16)
        xs_ref[base:base + HW, C:2 * C] = x_ref[n].astype(bf16)
    xs_ref[G:G + M, 0:C] = xs_ref[G + 1:G + 1 + M, C:2 * C] * m_notL
    xs_ref[G:G + M, 2 * C:3 * C] = xs_ref[G - 1:G - 1 + M, C:2 * C] * m_notR

    acc = jnp.dot(xs_ref[G - W:G - W + M, :], w1_ref[0],
                  preferred_element_type=f32)
    acc += jnp.dot(xs_ref[G:G + M, :], w1_ref[1], preferred_element_type=f32)
    acc += jnp.dot(xs_ref[G + W:G + W + M, :], w1_ref[2],
                   preferred_element_type=f32)
    y1 = jnp.maximum(acc * bn1s + bn1b, 0.0)

    ys_ref[G:G + M, C:2 * C] = y1.astype(bf16)
    for n in range(chunk):
        base = G + n * S
        ys_ref[base + HW:base + S, C:2 * C] = jnp.zeros((W, C), bf16)
    ys_ref[G:G + M, 0:C] = ys_ref[G + 1:G + 1 + M, C:2 * C] * m_notL
    ys_ref[G:G + M, 2 * C:3 * C] = ys_ref[G - 1:G - 1 + M, C:2 * C] * m_notR

    acc2 = jnp.dot(ys_ref[G - W:G - W + M, :], w2_ref[0],
                   preferred_element_type=f32)
    acc2 += jnp.dot(ys_ref[G:G + M, :], w2_ref[1], preferred_element_type=f32)
    acc2 += jnp.dot(ys_ref[G + W:G + W + M, :], w2_ref[2],
                    preferred_element_type=f32)
    r2 = acc2 * bn2s + bn2b

    pooled = jnp.concatenate(
        [jnp.sum(r2[n * S:n * S + HW, :], axis=0, keepdims=True)
         for n in range(chunk)], axis=0) * inv_hw
    hid = jnp.maximum(jnp.dot(pooled, fc1_ref[...],
                              preferred_element_type=f32), 0.0)
    att = jax.nn.sigmoid(jnp.dot(hid, fc2_ref[...], preferred_element_type=f32))

    for n in range(chunk):
        seg = r2[n * S:n * S + HW, :] * att[n:n + 1, :]
        o_ref[n] = jnp.maximum(seg + x_ref[n], 0.0)


@jax.jit
def _basic_block_att(x_nchw, w1, bn1_scale, bn1_bias, w2, bn2_scale, bn2_bias,
                     fc1, fc2):
    N, C, H, W = x_nchw.shape
    HW = H * W
    S = HW + W
    bf16 = jnp.bfloat16

    chunk = next(c for c in (8, 4, 2, 1) if N % c == 0 and N // c >= 2)
    n_steps = N // chunk
    M = chunk * S

    x = jnp.transpose(x_nchw, (0, 2, 3, 1)).reshape(N, HW, C)

    def pack(w):
        w = jnp.asarray(w)
        return jnp.stack([
            jnp.concatenate([w[kh, 2], w[kh, 1], w[kh, 0]], axis=0)
            for kh in range(3)]).astype(bf16)

    w1p = pack(w1)
    w2p = pack(w2)
    bn = jnp.stack([bn1_scale, bn1_bias, bn2_scale, bn2_bias]
                   ).astype(jnp.float32)
    fc1 = jnp.asarray(fc1, jnp.float32)
    fc2 = jnp.asarray(fc2, jnp.float32)
    Cr = fc1.shape[1]

    kfn = functools.partial(_fused_block_kernel, chunk=chunk, H=H, W=W, C=C,
                            inv_hw=1.0 / float(HW))

    out = pl.pallas_call(
        kfn,
        out_shape=jax.ShapeDtypeStruct((N, HW, C), jnp.float32),
        grid=(n_steps,),
        in_specs=[
            pl.BlockSpec((chunk, HW, C), lambda b: (b, 0, 0)),
            pl.BlockSpec((3, 3 * C, C), lambda b: (0, 0, 0)),
            pl.BlockSpec((3, 3 * C, C), lambda b: (0, 0, 0)),
            pl.BlockSpec((4, C), lambda b: (0, 0)),
            pl.BlockSpec((C, Cr), lambda b: (0, 0)),
            pl.BlockSpec((Cr, C), lambda b: (0, 0)),
        ],
        out_specs=pl.BlockSpec((chunk, HW, C), lambda b: (b, 0, 0)),
        scratch_shapes=[pltpu.VMEM((M + 64, 3 * C), bf16)],
        compiler_params=pltpu.CompilerParams(
            dimension_semantics=("parallel",),
            vmem_limit_bytes=48 * 1024 * 1024),
    )(x, w1p, w2p, bn, fc1, fc2)

    out = out.reshape(N, H, W, C)
    return jnp.transpose(out, (0, 3, 1, 2))


def kernel(x, w1, bn1_scale, bn1_bias, w2, bn2_scale, bn2_bias,
           ws, bns_scale, bns_bias, fc1, fc2):
    del ws, bns_scale, bns_bias
    return _basic_block_att(x, w1, bn1_scale, bn1_bias,
                            w2, bn2_scale, bn2_bias, fc1, fc2)
